# Initial kernel scaffold; baseline (speedup 1.0000x reference)
import jax
import jax.numpy as jnp
from jax.experimental import pallas as pl


def kernel(voxel_features, coords, batch_size):
    raise NotImplementedError("write your pallas kernel here")


# speedup vs baseline: 4.1956x; 4.1956x over previous
import functools

import jax
import jax.numpy as jnp
from jax import lax
from jax.experimental import pallas as pl
from jax.experimental.pallas import tpu as pltpu
from jax.experimental.pallas import tpu_sc as plsc

NY = 496
NX = 432
NCH = 64
B = 4
S = NY * NX
M = B * S
YG = 64
CP = 4000


def _sc_scatter(P):
  mesh = plsc.VectorSubcoreMesh(
      core_axis_name="c", subcore_axis_name="s", num_cores=2, num_subcores=16)
  nchunks_in = P // CP

  @functools.partial(
      pl.kernel,
      out_type=jax.ShapeDtypeStruct((B, NCH, NY, NX), jnp.float32),
      mesh=mesh,
      compiler_params=pltpu.CompilerParams(
          needs_layout_passes=False, use_tc_tiling_on_sc=True),
      scratch_types=[
          pltpu.VMEM((CP,), jnp.int32),
          pltpu.VMEM((CP,), jnp.int32),
          pltpu.VMEM((P + 960,), jnp.int32),
          pltpu.VMEM((P + 960,), jnp.int32),
          pltpu.VMEM((8, 8, NX), jnp.float32),
          pltpu.VMEM((17, 128), jnp.float32),
          pltpu.VMEM((16,), jnp.int32),
          pltpu.VMEM((32,), jnp.int32),
          pltpu.VMEM((32,), jnp.int32),
          pltpu.SemaphoreType.DMA,
      ],
  )
  def k(vf_hbm, tgt_hbm, pk_hbm, out_hbm,
        tgt_c, pk_c, pk_list, bk_list, slab, rows, idxb, starts, cur, sem):
    w = lax.axis_index("s") * 2 + lax.axis_index("c")
    bb = w >> 3
    gg = w & 7
    nyb = jnp.where(gg == 7, (NY - 7 * YG) // 8, YG // 8)
    iota = lax.iota(jnp.int32, 16)
    zero16i = jnp.zeros((16,), jnp.int32)
    zero16f = jnp.zeros((16,), jnp.float32)
    lane0 = iota == 0
    low8 = iota < 8

    def sload(ref, i):
      return ref[pl.ds(i, 16)][0]

    def sstore(ref, i, val):
      plsc.store_scatter(ref, [jnp.full((16,), i, jnp.int32)],
                         jnp.full((16,), val), mask=lane0)

    def chunk_body(ci, cnt):
      pltpu.sync_copy(tgt_hbm.at[pl.ds(ci * CP, CP)], tgt_c)
      pltpu.sync_copy(pk_hbm.at[pl.ds(ci * CP, CP)], pk_c)

      def vbody(v, cnt):
        t16 = tgt_c[pl.ds(v * 16, 16)]
        m = t16 == w
        mi = jnp.where(m, 1, 0).astype(jnp.int32)
        pos = cnt + plsc.cumsum(mi) - 1
        plsc.store_scatter(pk_list, [pos], pk_c[pl.ds(v * 16, 16)], mask=m)
        return cnt + jnp.sum(mi)

      return lax.fori_loop(0, CP // 16, vbody, cnt)

    k_t = lax.fori_loop(0, nchunks_in, chunk_body, jnp.int32(0))

    for q in range(2):
      starts[pl.ds(q * 16, 16)] = zero16i

    def h_body(i, _):
      v = sload(pk_list, i)
      yb = v >> 28
      sstore(starts, yb + 1, sload(starts, yb + 1) + 1)
      return 0

    lax.fori_loop(0, k_t, h_body, 0)

    def pf_body(t, _):
      sstore(starts, t + 1, sload(starts, t + 1) + sload(starts, t))
      return 0

    lax.fori_loop(0, 8, pf_body, 0)
    for q in range(2):
      cur[pl.ds(q * 16, 16)] = starts[pl.ds(q * 16, 16)]

    def s_body(i, _):
      v = sload(pk_list, i)
      yb = v >> 28
      pos = sload(cur, yb)
      sstore(bk_list, pos, v)
      sstore(cur, yb, pos + 1)
      return 0

    lax.fori_loop(0, k_t, s_body, 0)

    def zs_body(r, _):
      def zq_body(q, _):
        slab[r >> 3, r & 7, pl.ds(q * 16, 16)] = zero16f
        return 0

      lax.fori_loop(0, NX // 16, zq_body, 0)
      return 0

    lax.fori_loop(0, 64, zs_body, 0)

    for cw in range(8):
      def yb_body(t, _, cw=cw):
        lo = sload(starts, t)
        hi = sload(starts, t + 1)
        nck = (hi - lo + 15) >> 4

        def place_chunk(ck, _):
          j0 = lo + ck * 16
          lanes = j0 + iota
          valid = lanes < hi
          pv = bk_list[pl.ds(j0, 16)]
          pad = (w * 977 + j0 + iota * 131) & 16383
          idxb[pl.ds(0, 16)] = jnp.where(valid, (pv & 0xFFFF) >> 1, pad)
          pltpu.async_copy(
              vf_hbm.at[idxb], rows.at[pl.ds(0, 16)], sem).wait()
          for l in range(16):
            @pl.when(j0 + l < hi)
            def _():
              v = pv[l]
              xx = (v >> 16) & 511
              yl = (v >> 25) & 7
              half = (v & 1) * 64
              vals = rows[l, pl.ds(half + cw * 8, 16)]
              plsc.store_scatter(
                  slab,
                  [iota, jnp.full((16,), yl, jnp.int32),
                   jnp.full((16,), xx, jnp.int32)],
                  vals, mask=low8)
          return 0

        lax.fori_loop(0, nck, place_chunk, 0)
        pltpu.sync_copy(
            slab,
            out_hbm.at[bb, pl.ds(cw * 8, 8), pl.ds(gg * YG + t * 8, 8), :])

        def wipe_chunk(ck, _):
          j0 = lo + ck * 16
          pv = bk_list[pl.ds(j0, 16)]
          for l in range(16):
            @pl.when(j0 + l < hi)
            def _():
              v = pv[l]
              xx = (v >> 16) & 511
              yl = (v >> 25) & 7
              plsc.store_scatter(
                  slab,
                  [iota, jnp.full((16,), yl, jnp.int32),
                   jnp.full((16,), xx, jnp.int32)],
                  zero16f, mask=low8)
          return 0

        lax.fori_loop(0, nck, wipe_chunk, 0)
        return 0

      lax.fori_loop(0, nyb, yb_body, 0)

  return k


def kernel(voxel_features, coords, batch_size):
  P = voxel_features.shape[0]
  b = coords[:, 0].astype(jnp.int32)
  y = coords[:, 2].astype(jnp.int32)
  x = coords[:, 3].astype(jnp.int32)
  off = jnp.asarray(batch_size, jnp.int32) - B
  row = b * S + y * NX + x + off
  valid = (row >= 0) & (row < M)
  rowc = jnp.clip(row, 0, M - 1)
  b2 = rowc // S
  rem = rowc - b2 * S
  y2 = rem // NX
  x2 = rem - y2 * NX
  tgt = jnp.where(valid, b2 * 8 + (y2 >> 6), -1).astype(jnp.int32)
  p = jnp.arange(P, dtype=jnp.int32)
  pk = ((y2 & 63) << 25) | (x2 << 16) | p
  vf2 = voxel_features.reshape(P // 2, 2 * NCH)
  return _sc_scatter(P)(vf2, tgt, pk)


# speedup vs baseline: 5.3108x; 1.2658x over previous
import functools

import jax
import jax.numpy as jnp
from jax import lax
from jax.experimental import pallas as pl
from jax.experimental.pallas import tpu as pltpu
from jax.experimental.pallas import tpu_sc as plsc

NY = 496
NX = 432
NCH = 64
B = 4
S = NY * NX
M = B * S
YG = 64
CP = 4000


def _sc_scatter(P):
  mesh = plsc.VectorSubcoreMesh(
      core_axis_name="c", subcore_axis_name="s", num_cores=2, num_subcores=16)
  nchunks_in = P // CP

  @functools.partial(
      pl.kernel,
      out_type=jax.ShapeDtypeStruct((B, NCH, NY, NX), jnp.float32),
      mesh=mesh,
      compiler_params=pltpu.CompilerParams(
          needs_layout_passes=False, use_tc_tiling_on_sc=True),
      scratch_types=[
          pltpu.VMEM((CP,), jnp.int32),
          pltpu.VMEM((CP,), jnp.int32),
          pltpu.VMEM((CP + 96,), jnp.int32),
          pltpu.VMEM((P + 960,), jnp.int32),
          pltpu.VMEM((16, 8, NX), jnp.float32),
          pltpu.VMEM((17, 128), jnp.float32),
          pltpu.VMEM((16, 16), jnp.int32),
          pltpu.VMEM((16,), jnp.int32),
          pltpu.VMEM((48,), jnp.int32),
          pltpu.VMEM((48,), jnp.int32),
          pltpu.SemaphoreType.DMA,
      ],
  )
  def k(vf_hbm, tgt_hbm, pk_hbm, out_hbm,
        tgt_c, pk_c, cc, bk_list, slab, rows, h2, idxb, starts, cur, sem):
    w = lax.axis_index("s") * 2 + lax.axis_index("c")
    bb = w >> 3
    gg = w & 7
    nyb = jnp.where(gg == 7, (NY - 7 * YG) // 8, YG // 8)
    iota = lax.iota(jnp.int32, 16)
    zero16i = jnp.zeros((16,), jnp.int32)
    zero16f = jnp.zeros((16,), jnp.float32)
    one16i = jnp.ones((16,), jnp.int32)
    lane0 = iota == 0

    def sload(ref, i):
      return ref[pl.ds(i, 16)][0]

    def sstore(ref, i, val):
      plsc.store_scatter(ref, [jnp.full((16,), i, jnp.int32)],
                         jnp.full((16,), val), mask=lane0)

    for q in range(16):
      h2[q, pl.ds(0, 16)] = zero16i

    def ha_chunk(ci, _):
      pltpu.sync_copy(tgt_hbm.at[pl.ds(ci * CP, CP)], tgt_c)
      pltpu.sync_copy(pk_hbm.at[pl.ds(ci * CP, CP)], pk_c)

      def vbody(v, _):
        t16 = tgt_c[pl.ds(v * 16, 16)]
        m = t16 == w
        yb = (pk_c[pl.ds(v * 16, 16)] >> 28) & 7
        plsc.addupdate_scatter(h2, [iota, yb], one16i, mask=m)
        return 0

      return lax.fori_loop(0, CP // 16, vbody, 0)

    lax.fori_loop(0, nchunks_in, ha_chunk, 0)

    acc = zero16i
    for q in range(16):
      acc = acc + h2[q, pl.ds(0, 16)]
    starts[pl.ds(0, 16)] = zero16i
    starts[pl.ds(16, 16)] = zero16i
    starts[pl.ds(1, 16)] = plsc.cumsum(acc)

    def pf_fix(_, __):
      return 0

    for q in range(3):
      cur[pl.ds(q * 16, 16)] = starts[pl.ds(q * 16, 16)]

    def hb_chunk(ci, _):
      pltpu.sync_copy(tgt_hbm.at[pl.ds(ci * CP, CP)], tgt_c)
      pltpu.sync_copy(pk_hbm.at[pl.ds(ci * CP, CP)], pk_c)

      def vbody(v, cnt):
        t16 = tgt_c[pl.ds(v * 16, 16)]
        m = t16 == w
        mi = jnp.where(m, 1, 0).astype(jnp.int32)
        csum = plsc.cumsum(mi)
        pos = cnt + csum - 1
        plsc.store_scatter(cc, [pos], pk_c[pl.ds(v * 16, 16)], mask=m)
        return cnt + csum[15]

      ccn = lax.fori_loop(0, CP // 16, vbody, jnp.int32(0))

      def s_body(i, _):
        v = sload(cc, i)
        yb = (v >> 28) & 7
        pos = sload(cur, yb)
        sstore(bk_list, pos, v)
        sstore(cur, yb, pos + 1)
        return 0

      lax.fori_loop(0, ccn, s_body, 0)
      return 0

    lax.fori_loop(0, nchunks_in, hb_chunk, 0)

    def zs_body(r, _):
      def zq_body(q, _):
        slab[r >> 3, r & 7, pl.ds(q * 16, 16)] = zero16f
        return 0

      lax.fori_loop(0, NX // 16, zq_body, 0)
      return 0

    lax.fori_loop(0, 16 * 8, zs_body, 0)

    for cw in range(4):
      def yb_body(t, _, cw=cw):
        lo = sload(starts, t)
        hi = sload(starts, t + 1)
        nck = (hi - lo + 15) >> 4

        def place_chunk(ck, _):
          j0 = lo + ck * 16
          lanes = j0 + iota
          valid = lanes < hi
          pv = bk_list[pl.ds(j0, 16)]
          pad = (w * 977 + j0 + iota * 131) & 16383
          idxb[pl.ds(0, 16)] = jnp.where(valid, (pv & 0xFFFF) >> 1, pad)
          pltpu.async_copy(
              vf_hbm.at[idxb], rows.at[pl.ds(0, 16)], sem).wait()
          for l in range(16):
            @pl.when(j0 + l < hi)
            def _():
              v = pv[l]
              xx = (v >> 16) & 511
              yl = (v >> 25) & 7
              half = (v & 1) * 64
              vals = rows[l, pl.ds(half + cw * 16, 16)]
              plsc.store_scatter(
                  slab,
                  [iota, jnp.full((16,), yl, jnp.int32),
                   jnp.full((16,), xx, jnp.int32)],
                  vals)
          return 0

        lax.fori_loop(0, nck, place_chunk, 0)
        pltpu.sync_copy(
            slab,
            out_hbm.at[bb, pl.ds(cw * 16, 16), pl.ds(gg * YG + t * 8, 8), :])

        def wipe_chunk(ck, _):
          j0 = lo + ck * 16
          pv = bk_list[pl.ds(j0, 16)]
          for l in range(16):
            @pl.when(j0 + l < hi)
            def _():
              v = pv[l]
              xx = (v >> 16) & 511
              yl = (v >> 25) & 7
              plsc.store_scatter(
                  slab,
                  [iota, jnp.full((16,), yl, jnp.int32),
                   jnp.full((16,), xx, jnp.int32)],
                  zero16f)
          return 0

        lax.fori_loop(0, nck, wipe_chunk, 0)
        return 0

      lax.fori_loop(0, nyb, yb_body, 0)

  return k


def kernel(voxel_features, coords, batch_size):
  P = voxel_features.shape[0]
  b = coords[:, 0].astype(jnp.int32)
  y = coords[:, 2].astype(jnp.int32)
  x = coords[:, 3].astype(jnp.int32)
  off = jnp.asarray(batch_size, jnp.int32) - B
  row = b * S + y * NX + x + off
  valid = (row >= 0) & (row < M)
  rowc = jnp.clip(row, 0, M - 1)
  b2 = rowc // S
  rem = rowc - b2 * S
  y2 = rem // NX
  x2 = rem - y2 * NX
  tgt = jnp.where(valid, b2 * 8 + (y2 >> 6), -1).astype(jnp.int32)
  p = jnp.arange(P, dtype=jnp.int32)
  pk = ((y2 & 63) << 25) | (x2 << 16) | p
  vf2 = voxel_features.reshape(P // 2, 2 * NCH)
  return _sc_scatter(P)(vf2, tgt, pk)


# speedup vs baseline: 6.3968x; 1.2045x over previous
import functools

import jax
import jax.numpy as jnp
from jax import lax
from jax.experimental import pallas as pl
from jax.experimental.pallas import tpu as pltpu
from jax.experimental.pallas import tpu_sc as plsc

NY = 496
NX = 432
NCH = 64
B = 4
S = NY * NX
M = B * S
YG = 64
CP = 4000


def _sc_scatter(P):
  mesh = plsc.VectorSubcoreMesh(
      core_axis_name="c", subcore_axis_name="s", num_cores=2, num_subcores=16)
  nchunks_in = P // CP

  @functools.partial(
      pl.kernel,
      out_type=jax.ShapeDtypeStruct((B, NCH, NY, NX), jnp.float32),
      mesh=mesh,
      compiler_params=pltpu.CompilerParams(
          needs_layout_passes=False, use_tc_tiling_on_sc=True),
      scratch_types=[
          pltpu.VMEM((CP,), jnp.int32),
          pltpu.VMEM((CP,), jnp.int32),
          pltpu.VMEM((CP + 96,), jnp.int32),
          pltpu.VMEM((P + 960,), jnp.int32),
          pltpu.VMEM((16, 8, NX), jnp.float32),
          pltpu.VMEM((65, 128), jnp.float32),
          pltpu.VMEM((16, 16), jnp.int32),
          pltpu.VMEM((64,), jnp.int32),
          pltpu.VMEM((48,), jnp.int32),
          pltpu.VMEM((48,), jnp.int32),
          pltpu.SemaphoreType.DMA,
      ],
  )
  def k(vf_hbm, tgt_hbm, pk_hbm, out_hbm,
        tgt_c, pk_c, cc, bk_list, slab, rows, h2, idxb, starts, cur, sem):
    w = lax.axis_index("s") * 2 + lax.axis_index("c")
    bb = w >> 3
    gg = w & 7
    nyb = jnp.where(gg == 7, (NY - 7 * YG) // 8, YG // 8)
    iota = lax.iota(jnp.int32, 16)
    zero16i = jnp.zeros((16,), jnp.int32)
    zero16f = jnp.zeros((16,), jnp.float32)
    one16i = jnp.ones((16,), jnp.int32)
    lane0 = iota == 0

    def sload(ref, i):
      return ref[pl.ds(i, 16)][0]

    def sstore(ref, i, val):
      plsc.store_scatter(ref, [jnp.full((16,), i, jnp.int32)],
                         jnp.full((16,), val), mask=lane0)

    for q in range(16):
      h2[q, pl.ds(0, 16)] = zero16i

    def ha_chunk(ci, _):
      pltpu.sync_copy(tgt_hbm.at[pl.ds(ci * CP, CP)], tgt_c)
      pltpu.sync_copy(pk_hbm.at[pl.ds(ci * CP, CP)], pk_c)

      def vbody(v, _):
        t16 = tgt_c[pl.ds(v * 16, 16)]
        m = t16 == w
        yb = (pk_c[pl.ds(v * 16, 16)] >> 28) & 7
        plsc.addupdate_scatter(h2, [iota, yb], one16i, mask=m)
        return 0

      return lax.fori_loop(0, CP // 16, vbody, 0)

    lax.fori_loop(0, nchunks_in, ha_chunk, 0)

    acc = zero16i
    for q in range(16):
      acc = acc + h2[q, pl.ds(0, 16)]
    starts[pl.ds(0, 16)] = zero16i
    starts[pl.ds(16, 16)] = zero16i
    starts[pl.ds(1, 16)] = plsc.cumsum(acc)

    def pf_fix(_, __):
      return 0

    for q in range(3):
      cur[pl.ds(q * 16, 16)] = starts[pl.ds(q * 16, 16)]

    def hb_chunk(ci, _):
      pltpu.sync_copy(tgt_hbm.at[pl.ds(ci * CP, CP)], tgt_c)
      pltpu.sync_copy(pk_hbm.at[pl.ds(ci * CP, CP)], pk_c)

      def vbody(v, cnt):
        t16 = tgt_c[pl.ds(v * 16, 16)]
        m = t16 == w
        mi = jnp.where(m, 1, 0).astype(jnp.int32)
        csum = plsc.cumsum(mi)
        pos = cnt + csum - 1
        plsc.store_scatter(cc, [pos], pk_c[pl.ds(v * 16, 16)], mask=m)
        return cnt + csum[15]

      ccn = lax.fori_loop(0, CP // 16, vbody, jnp.int32(0))

      def s_body(i, _):
        v = sload(cc, i)
        yb = (v >> 28) & 7
        pos = sload(cur, yb)
        sstore(bk_list, pos, v)
        sstore(cur, yb, pos + 1)
        return 0

      lax.fori_loop(0, ccn, s_body, 0)
      return 0

    lax.fori_loop(0, nchunks_in, hb_chunk, 0)

    def zs_body(r, _):
      def zq_body(q, _):
        slab[r >> 3, r & 7, pl.ds(q * 16, 16)] = zero16f
        return 0

      lax.fori_loop(0, NX // 16, zq_body, 0)
      return 0

    lax.fori_loop(0, 16 * 8, zs_body, 0)

    for cw in range(4):
      def yb_body(t, _, cw=cw):
        lo = sload(starts, t)
        hi = sload(starts, t + 1)
        nck = (hi - lo + 63) >> 6

        def place_chunk(ck, _):
          j0 = lo + ck * 64

          def idx_q(q, _):
            j0q = j0 + q * 16
            pv = bk_list[pl.ds(j0q, 16)]
            valid = (j0q + iota) < hi
            pad = (w * 977 + j0q + iota * 131) & 16383
            idxb[pl.ds(q * 16, 16)] = jnp.where(
                valid, (pv & 0xFFFF) >> 1, pad)
            return 0

          lax.fori_loop(0, 4, idx_q, 0)
          pltpu.async_copy(
              vf_hbm.at[idxb], rows.at[pl.ds(0, 64)], sem).wait()

          def place_q(q, _):
            j0q = j0 + q * 16
            pv = bk_list[pl.ds(j0q, 16)]
            for l in range(16):
              @pl.when(j0q + l < hi)
              def _():
                v = pv[l]
                xx = (v >> 16) & 511
                yl = (v >> 25) & 7
                half = (v & 1) * 64
                vals = rows[q * 16 + l, pl.ds(half + cw * 16, 16)]
                plsc.store_scatter(
                    slab,
                    [iota, jnp.full((16,), yl, jnp.int32),
                     jnp.full((16,), xx, jnp.int32)],
                    vals)
            return 0

          lax.fori_loop(0, 4, place_q, 0)
          return 0

        lax.fori_loop(0, nck, place_chunk, 0)
        pltpu.sync_copy(
            slab,
            out_hbm.at[bb, pl.ds(cw * 16, 16), pl.ds(gg * YG + t * 8, 8), :])

        def wipe_chunk(ck, _):
          j0 = lo + ck * 64

          def wipe_q(q, _):
            j0q = j0 + q * 16
            pv = bk_list[pl.ds(j0q, 16)]
            for l in range(16):
              @pl.when(j0q + l < hi)
              def _():
                v = pv[l]
                xx = (v >> 16) & 511
                yl = (v >> 25) & 7
                plsc.store_scatter(
                    slab,
                    [iota, jnp.full((16,), yl, jnp.int32),
                     jnp.full((16,), xx, jnp.int32)],
                    zero16f)
            return 0

          lax.fori_loop(0, 4, wipe_q, 0)
          return 0

        lax.fori_loop(0, nck, wipe_chunk, 0)
        return 0

      lax.fori_loop(0, nyb, yb_body, 0)

  return k


def kernel(voxel_features, coords, batch_size):
  P = voxel_features.shape[0]
  b = coords[:, 0].astype(jnp.int32)
  y = coords[:, 2].astype(jnp.int32)
  x = coords[:, 3].astype(jnp.int32)
  off = jnp.asarray(batch_size, jnp.int32) - B
  row = b * S + y * NX + x + off
  valid = (row >= 0) & (row < M)
  rowc = jnp.clip(row, 0, M - 1)
  b2 = rowc // S
  rem = rowc - b2 * S
  y2 = rem // NX
  x2 = rem - y2 * NX
  tgt = jnp.where(valid, b2 * 8 + (y2 >> 6), -1).astype(jnp.int32)
  p = jnp.arange(P, dtype=jnp.int32)
  pk = ((y2 & 63) << 25) | (x2 << 16) | p
  vf2 = voxel_features.reshape(P // 2, 2 * NCH)
  return _sc_scatter(P)(vf2, tgt, pk)


# speedup vs baseline: 6.5405x; 1.0225x over previous
import functools

import jax
import jax.numpy as jnp
from jax import lax
from jax.experimental import pallas as pl
from jax.experimental.pallas import tpu as pltpu
from jax.experimental.pallas import tpu_sc as plsc

NY = 496
NX = 432
NCH = 64
B = 4
S = NY * NX
M = B * S
YG = 64
CP = 4000


def _sc_scatter(P):
  mesh = plsc.VectorSubcoreMesh(
      core_axis_name="c", subcore_axis_name="s", num_cores=2, num_subcores=16)
  nchunks_in = P // CP

  @functools.partial(
      pl.kernel,
      out_type=jax.ShapeDtypeStruct((B, NCH, NY, NX), jnp.float32),
      mesh=mesh,
      compiler_params=pltpu.CompilerParams(
          needs_layout_passes=False, use_tc_tiling_on_sc=True),
      scratch_types=[
          pltpu.VMEM((CP,), jnp.int32),
          pltpu.VMEM((CP,), jnp.int32),
          pltpu.VMEM((CP + 96,), jnp.int32),
          pltpu.VMEM((P + 960,), jnp.int32),
          pltpu.VMEM((16, 8, NX), jnp.float32),
          pltpu.VMEM((65, 128), jnp.float32),
          pltpu.VMEM((16, 16), jnp.int32),
          pltpu.VMEM((64,), jnp.int32),
          pltpu.VMEM((48,), jnp.int32),
          pltpu.VMEM((48,), jnp.int32),
          pltpu.SemaphoreType.DMA,
      ],
  )
  def k(vf_hbm, tgt_hbm, pk_hbm, out_hbm,
        tgt_c, pk_c, cc, bk_list, slab, rows, h2, idxb, starts, cur, sem):
    w = lax.axis_index("s") * 2 + lax.axis_index("c")
    bb = w >> 3
    gg = w & 7
    nyb = jnp.where(gg == 7, (NY - 7 * YG) // 8, YG // 8)
    iota = lax.iota(jnp.int32, 16)
    zero16i = jnp.zeros((16,), jnp.int32)
    zero16f = jnp.zeros((16,), jnp.float32)
    one16i = jnp.ones((16,), jnp.int32)
    lane0 = iota == 0

    def sload(ref, i):
      return ref[pl.ds(i, 16)][0]

    def sstore(ref, i, val):
      plsc.store_scatter(ref, [jnp.full((16,), i, jnp.int32)],
                         jnp.full((16,), val), mask=lane0)

    for q in range(16):
      h2[q, pl.ds(0, 16)] = zero16i

    def ha_chunk(ci, _):
      pltpu.sync_copy(tgt_hbm.at[pl.ds(ci * CP, CP)], tgt_c)
      pltpu.sync_copy(pk_hbm.at[pl.ds(ci * CP, CP)], pk_c)

      def vbody(v, _):
        t16 = tgt_c[pl.ds(v * 16, 16)]
        m = t16 == w
        yb = (pk_c[pl.ds(v * 16, 16)] >> 28) & 7
        plsc.addupdate_scatter(h2, [iota, yb], one16i, mask=m)
        return 0

      return lax.fori_loop(0, CP // 16, vbody, 0)

    lax.fori_loop(0, nchunks_in, ha_chunk, 0)

    acc = zero16i
    for q in range(16):
      acc = acc + h2[q, pl.ds(0, 16)]
    starts[pl.ds(0, 16)] = zero16i
    starts[pl.ds(16, 16)] = zero16i
    starts[pl.ds(1, 16)] = plsc.cumsum(acc)

    def pf_fix(_, __):
      return 0

    for q in range(3):
      cur[pl.ds(q * 16, 16)] = starts[pl.ds(q * 16, 16)]

    def hb_chunk(ci, _):
      pltpu.sync_copy(tgt_hbm.at[pl.ds(ci * CP, CP)], tgt_c)
      pltpu.sync_copy(pk_hbm.at[pl.ds(ci * CP, CP)], pk_c)

      def vbody(v, cnt):
        t16 = tgt_c[pl.ds(v * 16, 16)]
        m = t16 == w
        mi = jnp.where(m, 1, 0).astype(jnp.int32)
        csum = plsc.cumsum(mi)
        pos = cnt + csum - 1
        plsc.store_scatter(cc, [pos], pk_c[pl.ds(v * 16, 16)], mask=m)
        return cnt + csum[15]

      ccn = lax.fori_loop(0, CP // 16, vbody, jnp.int32(0))

      def s_vreg(g, _):
        i0 = g * 16
        ccv = cc[pl.ds(i0, 16)]
        for l in range(16):
          @pl.when(i0 + l < ccn)
          def _():
            v = ccv[l]
            yb = (v >> 28) & 7
            pos = sload(cur, yb)
            sstore(bk_list, pos, v)
            sstore(cur, yb, pos + 1)
        return 0

      lax.fori_loop(0, (ccn + 15) >> 4, s_vreg, 0)
      return 0

    lax.fori_loop(0, nchunks_in, hb_chunk, 0)

    def zs_body(r, _):
      def zq_body(q, _):
        slab[r >> 3, r & 7, pl.ds(q * 16, 16)] = zero16f
        return 0

      lax.fori_loop(0, NX // 16, zq_body, 0)
      return 0

    lax.fori_loop(0, 16 * 8, zs_body, 0)

    for cw in range(4):
      def yb_body(t, _, cw=cw):
        lo = sload(starts, t)
        hi = sload(starts, t + 1)
        nck = (hi - lo + 63) >> 6

        def place_chunk(ck, _):
          j0 = lo + ck * 64

          def idx_q(q, _):
            j0q = j0 + q * 16
            pv = bk_list[pl.ds(j0q, 16)]
            valid = (j0q + iota) < hi
            pad = (w * 977 + j0q + iota * 131) & 16383
            idxb[pl.ds(q * 16, 16)] = jnp.where(
                valid, (pv & 0xFFFF) >> 1, pad)
            return 0

          lax.fori_loop(0, 4, idx_q, 0)
          pltpu.async_copy(
              vf_hbm.at[idxb], rows.at[pl.ds(0, 64)], sem).wait()

          def place_q(q, _):
            j0q = j0 + q * 16
            pv = bk_list[pl.ds(j0q, 16)]
            for l in range(16):
              @pl.when(j0q + l < hi)
              def _():
                v = pv[l]
                xx = (v >> 16) & 511
                yl = (v >> 25) & 7
                half = (v & 1) * 64
                vals = rows[q * 16 + l, pl.ds(half + cw * 16, 16)]
                plsc.store_scatter(
                    slab,
                    [iota, jnp.full((16,), yl, jnp.int32),
                     jnp.full((16,), xx, jnp.int32)],
                    vals)
            return 0

          lax.fori_loop(0, 4, place_q, 0)
          return 0

        lax.fori_loop(0, nck, place_chunk, 0)
        pltpu.sync_copy(
            slab,
            out_hbm.at[bb, pl.ds(cw * 16, 16), pl.ds(gg * YG + t * 8, 8), :])

        def wipe_chunk(ck, _):
          j0 = lo + ck * 64

          def wipe_q(q, _):
            j0q = j0 + q * 16
            pv = bk_list[pl.ds(j0q, 16)]
            for l in range(16):
              @pl.when(j0q + l < hi)
              def _():
                v = pv[l]
                xx = (v >> 16) & 511
                yl = (v >> 25) & 7
                plsc.store_scatter(
                    slab,
                    [iota, jnp.full((16,), yl, jnp.int32),
                     jnp.full((16,), xx, jnp.int32)],
                    zero16f)
            return 0

          lax.fori_loop(0, 4, wipe_q, 0)
          return 0

        lax.fori_loop(0, nck, wipe_chunk, 0)
        return 0

      lax.fori_loop(0, nyb, yb_body, 0)

  return k


def kernel(voxel_features, coords, batch_size):
  P = voxel_features.shape[0]
  b = coords[:, 0].astype(jnp.int32)
  y = coords[:, 2].astype(jnp.int32)
  x = coords[:, 3].astype(jnp.int32)
  off = jnp.asarray(batch_size, jnp.int32) - B
  row = b * S + y * NX + x + off
  valid = (row >= 0) & (row < M)
  rowc = jnp.clip(row, 0, M - 1)
  b2 = rowc // S
  rem = rowc - b2 * S
  y2 = rem // NX
  x2 = rem - y2 * NX
  tgt = jnp.where(valid, b2 * 8 + (y2 >> 6), -1).astype(jnp.int32)
  p = jnp.arange(P, dtype=jnp.int32)
  pk = ((y2 & 63) << 25) | (x2 << 16) | p
  vf2 = voxel_features.reshape(P // 2, 2 * NCH)
  return _sc_scatter(P)(vf2, tgt, pk)
